# baseline (speedup 1.0000x reference)
import jax
import jax.numpy as jnp
import numpy as np
from jax import lax
from jax.experimental import pallas as pl
from jax.experimental.pallas import tpu as pltpu
from jax.experimental.pallas import tpu_sc as plsc

N = 10000
D = 128
NC = 2
NS = 16
NW = NC * NS
CHUNK = 96
NPAD = 10240
BR = 1024
ZROWS = 64
ROWS_ACC = NPAD // NS

E = 320000
ETOT = E + NPAD
EPT_CH = -(-ETOT // (NW * CHUNK))
EPT_CH += EPT_CH % 2
EPAD = EPT_CH * NW * CHUNK
EROWS = EPAD // CHUNK


def _deg_body(dst_hbm, deg_hbm, idxv, degl):
    cid = lax.axis_index("c")
    sid = lax.axis_index("s")
    wid = cid * NS + sid
    rpt = EROWS // NW

    def zero(i, c):
        degl[pl.ds(i * 16, 16)] = jnp.zeros((16,), jnp.float32)
        return c
    lax.fori_loop(0, NPAD // 16, zero, 0)

    pltpu.sync_copy(dst_hbm.at[pl.ds(wid * rpt, rpt)], idxv)

    ones = jnp.ones((16,), jnp.float32)

    def row(r, c):
        def col(k, c2):
            idx = idxv[r, pl.ds(k * 16, 16)]
            plsc.addupdate_scatter(degl, [idx], ones)
            return c2
        return lax.fori_loop(0, CHUNK // 16, col, c)
    lax.fori_loop(0, rpt, row, 0)

    pltpu.sync_copy(degl, deg_hbm.at[wid])


SEG = EROWS // NW
R0 = 122
R1 = 2 * SEG - R0

PERM = np.empty((D,), dtype=np.int32)
PERM[0::2] = np.arange(D // 2)
PERM[1::2] = D // 2 + np.arange(D // 2)
INV_PERM = np.argsort(PERM)


_MASKHI = -65536


def _agg_body(src_hbm, dst_hbm, g_hbm, out_hbm,
              srcv, dstv, buf0, buf1, fbuf, acc, sem0, sem1):
    cid = lax.axis_index("c")
    sid = lax.axis_index("s")

    def zr(r, c):
        def zc(k, c2):
            fbuf[r, pl.ds(k * 16, 16)] = jnp.zeros((16,), jnp.float32)
            return c2
        return lax.fori_loop(0, D // 16, zc, c)
    lax.fori_loop(0, ZROWS, zr, 0)

    base = sid * ROWS_ACC

    def za(k, c):
        pltpu.sync_copy(fbuf.at[pl.ds(0, ZROWS)],
                        acc.at[pl.ds(base + k * ZROWS, ZROWS)])
        return c
    lax.fori_loop(0, ROWS_ACC // ZROWS, za, 0)

    plsc.subcore_barrier()

    def convert(bi):
        def crow(r):
            for cg in range(D // 32):
                w = plsc.bitcast(bi[r, pl.ds(cg * 32, 32)], jnp.int32)
                fbuf[r, pl.ds(cg * 16, 16)] = plsc.bitcast(
                    w << 16, jnp.float32)
                fbuf[r, pl.ds(D // 2 + cg * 16, 16)] = plsc.bitcast(
                    w & _MASKHI, jnp.float32)
        plsc.parallel_loop(0, CHUNK, 1, unroll=8)(crow)

    def run_rows(row0, n):
        pltpu.sync_copy(src_hbm.at[pl.ds(row0, n)], srcv.at[pl.ds(0, n)])
        pltpu.sync_copy(dst_hbm.at[pl.ds(row0, n)], dstv.at[pl.ds(0, n)])
        pltpu.async_copy(g_hbm.at[srcv.at[0]], buf0, sem0)

        def step(jj, c):
            j0 = jj * 2
            pltpu.async_copy(g_hbm.at[srcv.at[j0 + 1]], buf1, sem1)
            pltpu.make_async_copy(g_hbm.at[srcv.at[j0]], buf0, sem0).wait()
            convert(buf0)
            pltpu.sync_copy(fbuf, acc.at[dstv.at[j0]], add=True)

            @pl.when(jj + 1 < n // 2)
            def _():
                pltpu.async_copy(g_hbm.at[srcv.at[j0 + 2]], buf0, sem0)

            pltpu.make_async_copy(g_hbm.at[srcv.at[j0 + 1]], buf1, sem1).wait()
            convert(buf1)
            pltpu.sync_copy(fbuf, acc.at[dstv.at[j0 + 1]], add=True)
            return c
        lax.fori_loop(0, n // 2, step, 0)

    def run_share(row0, n):
        if n > 0:
            run_rows(row0, min(n, SEG))
        if n > SEG:
            run_rows(row0 + SEG, n - SEG)

    if R0 > 0:
        @pl.when(cid == 0)
        def _():
            run_share(sid * R0, R0)

    if R1 > 0:
        @pl.when(cid == 1)
        def _():
            run_share(NS * R0 + sid * R1, R1)

    plsc.subcore_barrier()
    pltpu.sync_copy(acc.at[pl.ds(base, ROWS_ACC)],
                    out_hbm.at[pl.ds(cid * NPAD + base, ROWS_ACC)])


_MESH = plsc.VectorSubcoreMesh(core_axis_name="c", subcore_axis_name="s")

_SC_PARAMS = pltpu.CompilerParams(
    needs_layout_passes=False, use_tc_tiling_on_sc=False
)

_sc_deg = pl.kernel(
    _deg_body,
    out_type=jax.ShapeDtypeStruct((NW, NPAD), jnp.float32),
    mesh=_MESH,
    compiler_params=_SC_PARAMS,
    scratch_types=[
        pltpu.VMEM((EROWS // NW, CHUNK), jnp.int32),
        pltpu.VMEM((NPAD,), jnp.float32),
    ],
)

_sc_agg = pl.kernel(
    _agg_body,
    out_type=jax.ShapeDtypeStruct((NC * NPAD, D), jnp.float32),
    mesh=_MESH,
    compiler_params=_SC_PARAMS,
    scratch_types=[
        pltpu.VMEM((EROWS // NW, CHUNK), jnp.int32),
        pltpu.VMEM((EROWS // NW, CHUNK), jnp.int32),
        pltpu.VMEM((CHUNK, D), jnp.bfloat16),
        pltpu.VMEM((CHUNK, D), jnp.bfloat16),
        pltpu.VMEM((CHUNK, D), jnp.float32),
        pltpu.VMEM_SHARED((NPAD, D), jnp.float32),
        pltpu.SemaphoreType.DMA,
        pltpu.SemaphoreType.DMA,
    ],
)


def _dinv_col(degt):
    s = jnp.dot(degt, jnp.ones((NW, 1), jnp.float32),
                precision=lax.Precision.HIGHEST,
                preferred_element_type=jnp.float32)
    return lax.rsqrt(s)


def _tc1_body(degt_ref, x_ref, w_ref, g_ref):
    dinv = _dinv_col(degt_ref[...])
    h = jnp.dot(x_ref[...], w_ref[...],
                precision=lax.Precision.HIGHEST,
                preferred_element_type=jnp.float32)
    g_ref[...] = (dinv * h).astype(jnp.bfloat16)


def _tc2_body(degt_ref, p_ref, b_ref, w_ref, g2_ref):
    dinv = _dinv_col(degt_ref[...])
    agg = p_ref[0] + p_ref[1]
    z = jnp.maximum(dinv * agg + b_ref[...], 0.0)
    h = jnp.dot(z, w_ref[...],
                precision=lax.Precision.HIGHEST,
                preferred_element_type=jnp.float32)
    g2_ref[...] = (dinv * h).astype(jnp.bfloat16)


def _tc3_body(degt_ref, p_ref, b_ref, out_ref):
    dinv = _dinv_col(degt_ref[...])
    out_ref[...] = dinv * (p_ref[0] + p_ref[1]) + b_ref[...]


_GRID = (NPAD // BR,)

_DEGT_SPEC = pl.BlockSpec((BR, NW), lambda i: (i, 0))

_tc1 = pl.pallas_call(
    _tc1_body,
    grid=_GRID,
    in_specs=[
        _DEGT_SPEC,
        pl.BlockSpec((BR, D), lambda i: (i, 0)),
        pl.BlockSpec((D, D), lambda i: (0, 0)),
    ],
    out_specs=pl.BlockSpec((BR, D), lambda i: (i, 0)),
    out_shape=jax.ShapeDtypeStruct((NPAD, D), jnp.bfloat16),
)

_tc2 = pl.pallas_call(
    _tc2_body,
    grid=_GRID,
    in_specs=[
        _DEGT_SPEC,
        pl.BlockSpec((NC, BR, D), lambda i: (0, i, 0)),
        pl.BlockSpec((1, D), lambda i: (0, 0)),
        pl.BlockSpec((D, D), lambda i: (0, 0)),
    ],
    out_specs=pl.BlockSpec((BR, D), lambda i: (i, 0)),
    out_shape=jax.ShapeDtypeStruct((NPAD, D), jnp.bfloat16),
)

_tc3 = pl.pallas_call(
    _tc3_body,
    grid=_GRID,
    in_specs=[
        _DEGT_SPEC,
        pl.BlockSpec((NC, BR, D), lambda i: (0, i, 0)),
        pl.BlockSpec((1, D), lambda i: (0, 0)),
    ],
    out_specs=pl.BlockSpec((BR, D), lambda i: (i, 0)),
    out_shape=jax.ShapeDtypeStruct((NPAD, D), jnp.float32),
)


@jax.jit
def kernel(x, adj_t, W1, b1, W2, b2):
    x = x.astype(jnp.float32)
    src = adj_t[0].astype(jnp.int32)
    dst = adj_t[1].astype(jnp.int32)
    pad = EPAD - ETOT
    loops = jnp.arange(NPAD, dtype=jnp.int32)
    pad_dst = N + (jnp.arange(pad, dtype=jnp.int32) % (NPAD - N))
    src_p = jnp.concatenate(
        [src, loops, jnp.zeros((pad,), jnp.int32)]).reshape(EROWS, CHUNK)
    dst_p = jnp.concatenate([dst, loops, pad_dst]).reshape(EROWS, CHUNK)
    xp = jnp.zeros((NPAD, D), jnp.float32).at[:N].set(x)

    w1p = W1[:, PERM]
    w2p = W2[:, PERM]

    degt = _sc_deg(dst_p).T
    g1 = _tc1(degt, xp, w1p)
    p1 = _sc_agg(src_p, dst_p, g1).reshape(NC, NPAD, D)
    g2 = _tc2(degt, p1, b1.reshape(1, D), w2p)
    p2 = _sc_agg(src_p, dst_p, g2).reshape(NC, NPAD, D)
    out = _tc3(degt, p2, b2.reshape(1, D))
    return out[:N]


# speedup vs baseline: 1.0123x; 1.0123x over previous
import jax
import jax.numpy as jnp
import numpy as np
from jax import lax
from jax.experimental import pallas as pl
from jax.experimental.pallas import tpu as pltpu
from jax.experimental.pallas import tpu_sc as plsc

N = 10000
D = 128
NC = 2
NS = 16
NW = NC * NS
CHUNK = 96
NPAD = 10240
BR = 1024
ZROWS = 64
ROWS_ACC = NPAD // NS

E = 320000
ETOT = E + NPAD
EPT_CH = -(-ETOT // (NW * CHUNK))
EPT_CH += EPT_CH % 2
EPAD = EPT_CH * NW * CHUNK
EROWS = EPAD // CHUNK


def _deg_body(dst_hbm, deg_hbm, idxv, degl):
    cid = lax.axis_index("c")
    sid = lax.axis_index("s")
    wid = cid * NS + sid
    rpt = EROWS // NW

    def zero(i, c):
        degl[pl.ds(i * 16, 16)] = jnp.zeros((16,), jnp.float32)
        return c
    lax.fori_loop(0, NPAD // 16, zero, 0)

    pltpu.sync_copy(dst_hbm.at[pl.ds(wid * rpt, rpt)], idxv)

    ones = jnp.ones((16,), jnp.float32)

    def row(r, c):
        def col(k, c2):
            idx = idxv[r, pl.ds(k * 16, 16)]
            plsc.addupdate_scatter(degl, [idx], ones)
            return c2
        return lax.fori_loop(0, CHUNK // 16, col, c)
    lax.fori_loop(0, rpt, row, 0)

    pltpu.sync_copy(degl, deg_hbm.at[wid])


SEG = EROWS // NW
R0 = 116
R1 = 2 * SEG - R0

PERM = np.empty((D,), dtype=np.int32)
PERM[0::2] = np.arange(D // 2)
PERM[1::2] = D // 2 + np.arange(D // 2)
INV_PERM = np.argsort(PERM)


_MASKHI = -65536


def _agg_body(src_hbm, dst_hbm, g_hbm, out_hbm,
              srcv, dstv, buf0, buf1, fbuf, acc, sem0, sem1):
    cid = lax.axis_index("c")
    sid = lax.axis_index("s")

    def zr(r, c):
        def zc(k, c2):
            fbuf[r, pl.ds(k * 16, 16)] = jnp.zeros((16,), jnp.float32)
            return c2
        return lax.fori_loop(0, D // 16, zc, c)
    lax.fori_loop(0, ZROWS, zr, 0)

    base = sid * ROWS_ACC

    def za(k, c):
        pltpu.sync_copy(fbuf.at[pl.ds(0, ZROWS)],
                        acc.at[pl.ds(base + k * ZROWS, ZROWS)])
        return c
    lax.fori_loop(0, ROWS_ACC // ZROWS, za, 0)

    plsc.subcore_barrier()

    def convert(bi):
        def crow(r):
            for cg in range(D // 32):
                w = plsc.bitcast(bi[r, pl.ds(cg * 32, 32)], jnp.int32)
                fbuf[r, pl.ds(cg * 16, 16)] = plsc.bitcast(
                    w << 16, jnp.float32)
                fbuf[r, pl.ds(D // 2 + cg * 16, 16)] = plsc.bitcast(
                    w & _MASKHI, jnp.float32)
        plsc.parallel_loop(0, CHUNK, 1, unroll=8)(crow)

    def run_rows(row0, n):
        pltpu.sync_copy(src_hbm.at[pl.ds(row0, n)], srcv.at[pl.ds(0, n)])
        pltpu.sync_copy(dst_hbm.at[pl.ds(row0, n)], dstv.at[pl.ds(0, n)])
        pltpu.async_copy(g_hbm.at[srcv.at[0]], buf0, sem0)

        def step(jj, c):
            j0 = jj * 2
            pltpu.async_copy(g_hbm.at[srcv.at[j0 + 1]], buf1, sem1)
            pltpu.make_async_copy(g_hbm.at[srcv.at[j0]], buf0, sem0).wait()
            convert(buf0)
            pltpu.sync_copy(fbuf, acc.at[dstv.at[j0]], add=True)

            @pl.when(jj + 1 < n // 2)
            def _():
                pltpu.async_copy(g_hbm.at[srcv.at[j0 + 2]], buf0, sem0)

            pltpu.make_async_copy(g_hbm.at[srcv.at[j0 + 1]], buf1, sem1).wait()
            convert(buf1)
            pltpu.sync_copy(fbuf, acc.at[dstv.at[j0 + 1]], add=True)
            return c
        lax.fori_loop(0, n // 2, step, 0)

    def run_share(row0, n):
        if n > 0:
            run_rows(row0, min(n, SEG))
        if n > SEG:
            run_rows(row0 + SEG, n - SEG)

    if R0 > 0:
        @pl.when(cid == 0)
        def _():
            run_share(sid * R0, R0)

    if R1 > 0:
        @pl.when(cid == 1)
        def _():
            run_share(NS * R0 + sid * R1, R1)

    plsc.subcore_barrier()
    pltpu.sync_copy(acc.at[pl.ds(base, ROWS_ACC)],
                    out_hbm.at[pl.ds(cid * NPAD + base, ROWS_ACC)])


_MESH = plsc.VectorSubcoreMesh(core_axis_name="c", subcore_axis_name="s")

_SC_PARAMS = pltpu.CompilerParams(
    needs_layout_passes=False, use_tc_tiling_on_sc=False
)

_sc_deg = pl.kernel(
    _deg_body,
    out_type=jax.ShapeDtypeStruct((NW, NPAD), jnp.float32),
    mesh=_MESH,
    compiler_params=_SC_PARAMS,
    scratch_types=[
        pltpu.VMEM((EROWS // NW, CHUNK), jnp.int32),
        pltpu.VMEM((NPAD,), jnp.float32),
    ],
)

_sc_agg = pl.kernel(
    _agg_body,
    out_type=jax.ShapeDtypeStruct((NC * NPAD, D), jnp.float32),
    mesh=_MESH,
    compiler_params=_SC_PARAMS,
    scratch_types=[
        pltpu.VMEM((EROWS // NW, CHUNK), jnp.int32),
        pltpu.VMEM((EROWS // NW, CHUNK), jnp.int32),
        pltpu.VMEM((CHUNK, D), jnp.bfloat16),
        pltpu.VMEM((CHUNK, D), jnp.bfloat16),
        pltpu.VMEM((CHUNK, D), jnp.float32),
        pltpu.VMEM_SHARED((NPAD, D), jnp.float32),
        pltpu.SemaphoreType.DMA,
        pltpu.SemaphoreType.DMA,
    ],
)


def _dinv_col(degt):
    s = jnp.dot(degt, jnp.ones((NW, 1), jnp.float32),
                precision=lax.Precision.HIGHEST,
                preferred_element_type=jnp.float32)
    return lax.rsqrt(s)


def _tc1_body(degt_ref, x_ref, w_ref, g_ref):
    dinv = _dinv_col(degt_ref[...])
    h = jnp.dot(x_ref[...], w_ref[...],
                precision=lax.Precision.HIGHEST,
                preferred_element_type=jnp.float32)
    g_ref[...] = (dinv * h).astype(jnp.bfloat16)


def _tc2_body(degt_ref, p_ref, b_ref, w_ref, g2_ref):
    dinv = _dinv_col(degt_ref[...])
    agg = p_ref[0] + p_ref[1]
    z = jnp.maximum(dinv * agg + b_ref[...], 0.0)
    h = jnp.dot(z, w_ref[...],
                precision=lax.Precision.HIGHEST,
                preferred_element_type=jnp.float32)
    g2_ref[...] = (dinv * h).astype(jnp.bfloat16)


def _tc3_body(degt_ref, p_ref, b_ref, out_ref):
    dinv = _dinv_col(degt_ref[...])
    out_ref[...] = dinv * (p_ref[0] + p_ref[1]) + b_ref[...]


_GRID = (NPAD // BR,)

_DEGT_SPEC = pl.BlockSpec((BR, NW), lambda i: (i, 0))

_tc1 = pl.pallas_call(
    _tc1_body,
    grid=_GRID,
    in_specs=[
        _DEGT_SPEC,
        pl.BlockSpec((BR, D), lambda i: (i, 0)),
        pl.BlockSpec((D, D), lambda i: (0, 0)),
    ],
    out_specs=pl.BlockSpec((BR, D), lambda i: (i, 0)),
    out_shape=jax.ShapeDtypeStruct((NPAD, D), jnp.bfloat16),
)

_tc2 = pl.pallas_call(
    _tc2_body,
    grid=_GRID,
    in_specs=[
        _DEGT_SPEC,
        pl.BlockSpec((NC, BR, D), lambda i: (0, i, 0)),
        pl.BlockSpec((1, D), lambda i: (0, 0)),
        pl.BlockSpec((D, D), lambda i: (0, 0)),
    ],
    out_specs=pl.BlockSpec((BR, D), lambda i: (i, 0)),
    out_shape=jax.ShapeDtypeStruct((NPAD, D), jnp.bfloat16),
)

_tc3 = pl.pallas_call(
    _tc3_body,
    grid=_GRID,
    in_specs=[
        _DEGT_SPEC,
        pl.BlockSpec((NC, BR, D), lambda i: (0, i, 0)),
        pl.BlockSpec((1, D), lambda i: (0, 0)),
    ],
    out_specs=pl.BlockSpec((BR, D), lambda i: (i, 0)),
    out_shape=jax.ShapeDtypeStruct((NPAD, D), jnp.float32),
)


@jax.jit
def kernel(x, adj_t, W1, b1, W2, b2):
    x = x.astype(jnp.float32)
    src = adj_t[0].astype(jnp.int32)
    dst = adj_t[1].astype(jnp.int32)
    pad = EPAD - ETOT
    loops = jnp.arange(NPAD, dtype=jnp.int32)
    pad_dst = N + (jnp.arange(pad, dtype=jnp.int32) % (NPAD - N))
    src_p = jnp.concatenate(
        [src, loops, jnp.zeros((pad,), jnp.int32)]).reshape(EROWS, CHUNK)
    dst_p = jnp.concatenate([dst, loops, pad_dst]).reshape(EROWS, CHUNK)
    xp = jnp.zeros((NPAD, D), jnp.float32).at[:N].set(x)

    w1p = W1[:, PERM]
    w2p = W2[:, PERM]

    degt = _sc_deg(dst_p).T
    g1 = _tc1(degt, xp, w1p)
    p1 = _sc_agg(src_p, dst_p, g1).reshape(NC, NPAD, D)
    g2 = _tc2(degt, p1, b1.reshape(1, D), w2p)
    p2 = _sc_agg(src_p, dst_p, g2).reshape(NC, NPAD, D)
    out = _tc3(degt, p2, b2.reshape(1, D))
    return out[:N]
